# Initial kernel scaffold; baseline (speedup 1.0000x reference)
import jax
import jax.numpy as jnp
from jax.experimental import pallas as pl


def kernel(feat, lposition, edge_index, W, w_u, w_v, w_s, beta, aw, theta):
    raise NotImplementedError("write your pallas kernel here")


# speedup vs baseline: 3.7740x; 3.7740x over previous
import functools

import jax
import jax.numpy as jnp
from jax import lax
from jax.experimental import pallas as pl
from jax.experimental.pallas import tpu as pltpu
from jax.experimental.pallas import tpu_sc as plsc

N = 10000
E = 320000
D = 128
NC = 2
NS = 16
NW = NC * NS
EW = E // NW
CH = 80
G = CH // 16
NCHUNK = EW // CH
NP = 10240
STR = NP // NS
NEG_SLOPE = 0.2

_mesh = plsc.VectorSubcoreMesh(core_axis_name="c", subcore_axis_name="s")


def _tc1_body(feat_ref, lp_ref, w_ref, wu_ref, wv_ref, ws_ref, beta_ref,
              aw_ref, f_ref, b_ref):
    fs = lax.dot_general(feat_ref[...], w_ref[...], (((1,), (1,)), ((), ())),
                         preferred_element_type=jnp.float32)
    lr = jnp.where(fs > 0, fs, NEG_SLOPE * fs)
    el = jnp.sum(lr * wu_ref[...], axis=1, keepdims=True)
    er = jnp.sum(lr * wv_ref[...], axis=1, keepdims=True)
    ssf = jnp.sum(fs * fs, axis=1, keepdims=True)
    lp = lp_ref[...]
    ssl = jnp.sum(lp * lp, axis=1, keepdims=True)
    b = beta_ref[0, 0]
    betaw = 2.0 / (jnp.exp(-b) + 1.0)
    a0 = aw_ref[0, 0]
    a1 = aw_ref[0, 1]
    m = jnp.maximum(a0, a1)
    e0 = jnp.exp(a0 - m)
    e1 = jnp.exp(a1 - m)
    w0 = e0 / (e0 + e1)
    w1 = e1 / (e0 + e1)
    alpha = 2.0 * betaw * w0
    sqa = jnp.sqrt(alpha)
    cs = el - betaw * (w0 * ssf + w1 * ssl)
    cd = er - betaw * (w0 * ssf + w1 * ssl)
    f_ref[...] = fs * sqa
    gam = ws_ref[...] + 2.0 * betaw * w1
    br = lp.shape[0]
    one = jnp.ones((br, 1), jnp.float32)
    zpad = jnp.zeros((br, 46), jnp.float32)
    b_ref[...] = jnp.concatenate(
        [lp * gam, cs, one, zpad, lp, one, cd, zpad], axis=1)


def _tc1(feat, lposition, W, w_u, w_v, w_s, beta, aw):
    BR = 1000
    grid = (N // BR,)
    return pl.pallas_call(
        _tc1_body,
        grid=grid,
        in_specs=[
            pl.BlockSpec((BR, D), lambda i: (i, 0)),
            pl.BlockSpec((BR, 16), lambda i: (i, 0)),
            pl.BlockSpec((D, D), lambda i: (0, 0)),
            pl.BlockSpec((1, D), lambda i: (0, 0)),
            pl.BlockSpec((1, D), lambda i: (0, 0)),
            pl.BlockSpec((1, 16), lambda i: (0, 0)),
            pl.BlockSpec((1, 1), lambda i: (0, 0)),
            pl.BlockSpec((1, 2), lambda i: (0, 0)),
        ],
        out_specs=[
            pl.BlockSpec((BR, D), lambda i: (i, 0)),
            pl.BlockSpec((BR, D), lambda i: (i, 0)),
        ],
        out_shape=[
            jax.ShapeDtypeStruct((N, D), jnp.float32),
            jax.ShapeDtypeStruct((N, D), jnp.float32),
        ],
    )(feat, lposition, W, w_u, w_v, w_s, beta, aw)


def _sc1_body(f_hbm, b_hbm, src_hbm, dst_hbm,
              ew_hbm, degs_hbm, degd_hbm, ind_hbm,
              idx_s, idx_d, rows_s, rows_d, b_s, b_d, ewb, ones, zb,
              degs_sp, degd_sp, ind_sp, sem):
    c = lax.axis_index("c")
    s = lax.axis_index("s")
    wid = c * NS + s
    for i in range(STR // 16):
        zb[pl.ds(16 * i, 16)] = jnp.zeros((16,), jnp.float32)
    for i in range(G):
        ones[pl.ds(16 * i, 16)] = jnp.full((16,), 1.0, jnp.float32)
    pltpu.sync_copy(zb, degs_sp.at[pl.ds(s * STR, STR)])
    pltpu.sync_copy(zb, degd_sp.at[pl.ds(s * STR, STR)])
    pltpu.sync_copy(zb, ind_sp.at[pl.ds(s * STR, STR)])
    plsc.subcore_barrier()

    def chunk(i, carry):
        base = pl.multiple_of(wid * EW + i * CH, 8)
        pltpu.sync_copy(src_hbm.at[pl.ds(base, CH)], idx_s)
        pltpu.sync_copy(dst_hbm.at[pl.ds(base, CH)], idx_d)
        cp1 = pltpu.async_copy(f_hbm.at[idx_s], rows_s, sem)
        cp2 = pltpu.async_copy(f_hbm.at[idx_d], rows_d, sem)
        cp3 = pltpu.async_copy(b_hbm.at[idx_s], b_s, sem)
        cp4 = pltpu.async_copy(b_hbm.at[idx_d], b_d, sem)
        cp1.wait()
        cp2.wait()
        cp3.wait()
        cp4.wait()
        for g in range(G):
            ri = lax.iota(jnp.int32, 16) + g * 16
            acc = jnp.zeros((16,), jnp.float32)
            for k in range(D):
                ck = jnp.full((16,), k, jnp.int32)
                acc = acc + (plsc.load_gather(rows_s, [ri, ck]) *
                             plsc.load_gather(rows_d, [ri, ck]))
            for k in range(18):
                cks = jnp.full((16,), k, jnp.int32)
                ckd = jnp.full((16,), 64 + k, jnp.int32)
                acc = acc + (plsc.load_gather(b_s, [ri, cks]) *
                             plsc.load_gather(b_d, [ri, ckd]))
            ewb[pl.ds(g * 16, 16)] = jnp.exp(acc) + 1e-9
        pltpu.sync_copy(ewb, ew_hbm.at[pl.ds(base, CH)])
        pltpu.sync_copy(ewb, degs_sp.at[idx_s], add=True)
        pltpu.sync_copy(ewb, degd_sp.at[idx_d], add=True)
        pltpu.sync_copy(ones, ind_sp.at[idx_d], add=True)
        return carry

    lax.fori_loop(0, NCHUNK, chunk, 0)
    plsc.subcore_barrier()
    pltpu.sync_copy(degs_sp.at[pl.ds(s * STR, STR)],
                    degs_hbm.at[c, pl.ds(s * STR, STR)])
    pltpu.sync_copy(degd_sp.at[pl.ds(s * STR, STR)],
                    degd_hbm.at[c, pl.ds(s * STR, STR)])
    pltpu.sync_copy(ind_sp.at[pl.ds(s * STR, STR)],
                    ind_hbm.at[c, pl.ds(s * STR, STR)])


def _sc1(F, B, src, dst):
    return pl.kernel(
        _sc1_body,
        out_type=[
            jax.ShapeDtypeStruct((E,), jnp.float32),
            jax.ShapeDtypeStruct((NC, NP), jnp.float32),
            jax.ShapeDtypeStruct((NC, NP), jnp.float32),
            jax.ShapeDtypeStruct((NC, NP), jnp.float32),
        ],
        mesh=_mesh,
        compiler_params=pltpu.CompilerParams(needs_layout_passes=False),
        scratch_types=[
            pltpu.VMEM((CH,), jnp.int32),
            pltpu.VMEM((CH,), jnp.int32),
            pltpu.VMEM((CH, D), jnp.float32),
            pltpu.VMEM((CH, D), jnp.float32),
            pltpu.VMEM((CH, D), jnp.float32),
            pltpu.VMEM((CH, D), jnp.float32),
            pltpu.VMEM((CH,), jnp.float32),
            pltpu.VMEM((CH,), jnp.float32),
            pltpu.VMEM((STR,), jnp.float32),
            pltpu.VMEM_SHARED((NP,), jnp.float32),
            pltpu.VMEM_SHARED((NP,), jnp.float32),
            pltpu.VMEM_SHARED((NP,), jnp.float32),
            pltpu.SemaphoreType.DMA,
        ],
    )(F, B, src, dst)


def _tc2_body(degs_ref, degd_ref, ind_ref, beta_ref, aw_ref, theta_ref,
              ns2_ref, nd_ref, ps_ref):
    b = beta_ref[0, 0]
    betaw = 2.0 / (jnp.exp(-b) + 1.0)
    a0 = aw_ref[0, 0]
    a1 = aw_ref[0, 1]
    m = jnp.maximum(a0, a1)
    e0 = jnp.exp(a0 - m)
    e1 = jnp.exp(a1 - m)
    w0 = e0 / (e0 + e1)
    sqa = jnp.sqrt(2.0 * betaw * w0)
    degs = degs_ref[0, :] + degs_ref[1, :]
    degd = degd_ref[0, :] + degd_ref[1, :]
    ind = ind_ref[0, :] + ind_ref[1, :]
    ns2_ref[...] = lax.rsqrt(degs) / sqa
    nd_ref[...] = lax.rsqrt(degd)
    perm = 1e-9 / (jnp.exp(-theta_ref[0, 0]) + 1.0)
    ps_ref[...] = (perm / (ind + 1e-9)) / sqa


def _tc2(degs, degd, ind, beta, aw, theta):
    return pl.pallas_call(
        _tc2_body,
        out_shape=[
            jax.ShapeDtypeStruct((NP,), jnp.float32),
            jax.ShapeDtypeStruct((NP,), jnp.float32),
            jax.ShapeDtypeStruct((NP,), jnp.float32),
        ],
    )(degs, degd, ind, beta, aw, theta)


def _sc2_body(f_hbm, src_hbm, dst_hbm, ew_hbm, ns2_hbm, nd_hbm,
              rstp_hbm,
              idx_s, idx_d, rows, rows2, ewb, nsb, ndb, ab,
              rst_sp, sem):
    c = lax.axis_index("c")
    s = lax.axis_index("s")
    wid = c * NS + s
    for r in range(CH):
        for cgrp in range(D // 16):
            rows2[r, pl.ds(cgrp * 16, 16)] = jnp.zeros((16,), jnp.float32)
    for j in range(8):
        off = s * 640 + j * CH

        @pl.when(off < N)
        def _():
            pltpu.sync_copy(rows2,
                            rst_sp.at[pl.ds(pl.multiple_of(off, 8), CH)])
    plsc.subcore_barrier()

    def chunk(i, carry):
        base = pl.multiple_of(wid * EW + i * CH, 8)
        pltpu.sync_copy(src_hbm.at[pl.ds(base, CH)], idx_s)
        pltpu.sync_copy(dst_hbm.at[pl.ds(base, CH)], idx_d)
        cp1 = pltpu.async_copy(f_hbm.at[idx_s], rows, sem)
        cp2 = pltpu.async_copy(ns2_hbm.at[idx_s], nsb, sem)
        cp3 = pltpu.async_copy(nd_hbm.at[idx_d], ndb, sem)
        pltpu.sync_copy(ew_hbm.at[pl.ds(base, CH)], ewb)
        cp1.wait()
        cp2.wait()
        cp3.wait()
        for g in range(G):
            sl = pl.ds(g * 16, 16)
            ab[sl] = nsb[sl] * ndb[sl] * ewb[sl]
        for g in range(G):
            ri = lax.iota(jnp.int32, 16) + g * 16
            av = ab[pl.ds(g * 16, 16)]
            for k in range(D):
                ck = jnp.full((16,), k, jnp.int32)
                v = plsc.load_gather(rows, [ri, ck]) * av
                plsc.store_scatter(rows2, [ri, ck], v)
        pltpu.sync_copy(rows2, rst_sp.at[idx_d], add=True)
        return carry

    lax.fori_loop(0, NCHUNK, chunk, 0)
    plsc.subcore_barrier()
    for j in range(8):
        off = s * 640 + j * CH

        @pl.when(off < N)
        def _():
            o8 = pl.multiple_of(off, 8)
            pltpu.sync_copy(rst_sp.at[pl.ds(o8, CH)],
                            rstp_hbm.at[pl.ds(pl.multiple_of(c * N + o8, 8),
                                              CH)])


def _sc2(F, src, dst, ew, ns2, nd):
    return pl.kernel(
        _sc2_body,
        out_type=jax.ShapeDtypeStruct((NC * N, D), jnp.float32),
        mesh=_mesh,
        compiler_params=pltpu.CompilerParams(needs_layout_passes=False),
        scratch_types=[
            pltpu.VMEM((CH,), jnp.int32),
            pltpu.VMEM((CH,), jnp.int32),
            pltpu.VMEM((CH, D), jnp.float32),
            pltpu.VMEM((CH, D), jnp.float32),
            pltpu.VMEM((CH,), jnp.float32),
            pltpu.VMEM((CH,), jnp.float32),
            pltpu.VMEM((CH,), jnp.float32),
            pltpu.VMEM((CH,), jnp.float32),
            pltpu.VMEM_SHARED((N, D), jnp.float32),
            pltpu.SemaphoreType.DMA,
        ],
    )(F, src, dst, ew, ns2, nd)


def _tc3_body(r0_ref, r1_ref, f_ref, ps_ref, out_ref):
    x = r0_ref[...] + r1_ref[...] + f_ref[...] * ps_ref[...]
    out_ref[...] = jnp.where(x > 0, x, jnp.exp(x) - 1.0)


def _tc3(rst0, rst1, F, ps):
    BR = 1000
    return pl.pallas_call(
        _tc3_body,
        grid=(N // BR,),
        in_specs=[
            pl.BlockSpec((BR, D), lambda i: (i, 0)),
            pl.BlockSpec((BR, D), lambda i: (i, 0)),
            pl.BlockSpec((BR, D), lambda i: (i, 0)),
            pl.BlockSpec((BR, 1), lambda i: (i, 0)),
        ],
        out_specs=pl.BlockSpec((BR, D), lambda i: (i, 0)),
        out_shape=jax.ShapeDtypeStruct((N, D), jnp.float32),
    )(rst0, rst1, F, ps)


def kernel(feat, lposition, edge_index, W, w_u, w_v, w_s, beta, aw, theta):
    src = edge_index[0]
    dst = edge_index[1]
    wu = w_u.reshape(1, D)
    wv = w_v.reshape(1, D)
    ws = w_s.reshape(1, 16)
    F, B = _tc1(feat, lposition, W, wu, wv, ws, beta, aw)
    ew, degs, degd, ind = _sc1(F, B, src, dst)
    ns2, nd, ps = _tc2(degs, degd, ind, beta, aw, theta)
    rstp = _sc2(F, src, dst, ew, ns2, nd)
    out = _tc3(rstp[:N], rstp[N:], F, ps[:N].reshape(N, 1))
    return out


# speedup vs baseline: 4.0649x; 1.0771x over previous
import jax
import jax.numpy as jnp
from jax import lax
from jax.experimental import pallas as pl
from jax.experimental.pallas import tpu as pltpu
from jax.experimental.pallas import tpu_sc as plsc

N = 10000
E = 320000
D = 128
NC = 2
NS = 16
NW = NC * NS
EW = E // NW
CH = 80
G = CH // 16
NCHUNK = EW // CH
NPAIR = (NCHUNK - 1) // 2
NP = 10240
STR = NP // NS
NEG_SLOPE = 0.2

_mesh = plsc.VectorSubcoreMesh(core_axis_name="c", subcore_axis_name="s")
_sc_params = pltpu.CompilerParams(needs_layout_passes=False)


def _tc1_body(feat_ref, lp_ref, w_ref, wu_ref, wv_ref, ws_ref, beta_ref,
              aw_ref, f_ref, b_ref):
    fs = lax.dot_general(feat_ref[...], w_ref[...], (((1,), (1,)), ((), ())),
                         preferred_element_type=jnp.float32)
    lr = jnp.where(fs > 0, fs, NEG_SLOPE * fs)
    el = jnp.sum(lr * wu_ref[...], axis=1, keepdims=True)
    er = jnp.sum(lr * wv_ref[...], axis=1, keepdims=True)
    ssf = jnp.sum(fs * fs, axis=1, keepdims=True)
    lp = lp_ref[...]
    ssl = jnp.sum(lp * lp, axis=1, keepdims=True)
    b = beta_ref[0, 0]
    betaw = 2.0 / (jnp.exp(-b) + 1.0)
    a0 = aw_ref[0, 0]
    a1 = aw_ref[0, 1]
    m = jnp.maximum(a0, a1)
    e0 = jnp.exp(a0 - m)
    e1 = jnp.exp(a1 - m)
    w0 = e0 / (e0 + e1)
    w1 = e1 / (e0 + e1)
    alpha = 2.0 * betaw * w0
    sqa = jnp.sqrt(alpha)
    cs = el - betaw * (w0 * ssf + w1 * ssl)
    cd = er - betaw * (w0 * ssf + w1 * ssl)
    f_ref[...] = fs * sqa
    gam = ws_ref[...] + 2.0 * betaw * w1
    br = lp.shape[0]
    one = jnp.ones((br, 1), jnp.float32)
    zpad = jnp.zeros((br, 46), jnp.float32)
    b_ref[...] = jnp.concatenate(
        [lp * gam, cs, one, zpad, lp, one, cd, zpad], axis=1)


def _tc1(feat, lposition, W, w_u, w_v, w_s, beta, aw):
    BR = 1000
    grid = (N // BR,)
    return pl.pallas_call(
        _tc1_body,
        grid=grid,
        in_specs=[
            pl.BlockSpec((BR, D), lambda i: (i, 0)),
            pl.BlockSpec((BR, 16), lambda i: (i, 0)),
            pl.BlockSpec((D, D), lambda i: (0, 0)),
            pl.BlockSpec((1, D), lambda i: (0, 0)),
            pl.BlockSpec((1, D), lambda i: (0, 0)),
            pl.BlockSpec((1, 16), lambda i: (0, 0)),
            pl.BlockSpec((1, 1), lambda i: (0, 0)),
            pl.BlockSpec((1, 2), lambda i: (0, 0)),
        ],
        out_specs=[
            pl.BlockSpec((BR, D), lambda i: (i, 0)),
            pl.BlockSpec((BR, D), lambda i: (i, 0)),
        ],
        out_shape=[
            jax.ShapeDtypeStruct((N, D), jnp.float32),
            jax.ShapeDtypeStruct((N, D), jnp.float32),
        ],
    )(feat, lposition, W, w_u, w_v, w_s, beta, aw)


def _sc1_body(f_hbm, b_hbm, src_hbm, dst_hbm,
              ew_hbm, degs_hbm, degd_hbm, ind_hbm,
              idx_s, idx_d, rows_s, rows_d, b_s, b_d, ewb, ones, zb,
              degs_sp, degd_sp, ind_sp, sem0, sem1):
    c = lax.axis_index("c")
    s = lax.axis_index("s")
    wid = c * NS + s
    sems = (sem0, sem1)
    for i in range(STR // 16):
        zb[pl.ds(16 * i, 16)] = jnp.zeros((16,), jnp.float32)
    for i in range(G):
        ones[pl.ds(16 * i, 16)] = jnp.full((16,), 1.0, jnp.float32)
    pltpu.sync_copy(zb, degs_sp.at[pl.ds(s * STR, STR)])
    pltpu.sync_copy(zb, degd_sp.at[pl.ds(s * STR, STR)])
    pltpu.sync_copy(zb, ind_sp.at[pl.ds(s * STR, STR)])
    plsc.subcore_barrier()

    def fire(i, bk):
        base = pl.multiple_of(wid * EW + i * CH, 8)
        pltpu.sync_copy(src_hbm.at[pl.ds(base, CH)], idx_s.at[bk])
        pltpu.sync_copy(dst_hbm.at[pl.ds(base, CH)], idx_d.at[bk])
        pltpu.async_copy(f_hbm.at[idx_s.at[bk]], rows_s.at[bk], sems[bk])
        pltpu.async_copy(f_hbm.at[idx_d.at[bk]], rows_d.at[bk], sems[bk])
        pltpu.async_copy(b_hbm.at[idx_s.at[bk]], b_s.at[bk], sems[bk])
        pltpu.async_copy(b_hbm.at[idx_d.at[bk]], b_d.at[bk], sems[bk])

    def drain(bk):
        pltpu.make_async_copy(f_hbm.at[idx_s.at[bk]], rows_s.at[bk],
                              sems[bk]).wait()
        pltpu.make_async_copy(f_hbm.at[idx_d.at[bk]], rows_d.at[bk],
                              sems[bk]).wait()
        pltpu.make_async_copy(b_hbm.at[idx_s.at[bk]], b_s.at[bk],
                              sems[bk]).wait()
        pltpu.make_async_copy(b_hbm.at[idx_d.at[bk]], b_d.at[bk],
                              sems[bk]).wait()

    def compute(i, bk):
        base = pl.multiple_of(wid * EW + i * CH, 8)
        for g in range(G):
            ri = lax.iota(jnp.int32, 16) + g * 16
            acc = jnp.zeros((16,), jnp.float32)
            for k in range(D):
                ck = jnp.full((16,), k, jnp.int32)
                acc = acc + (plsc.load_gather(rows_s.at[bk], [ri, ck]) *
                             plsc.load_gather(rows_d.at[bk], [ri, ck]))
            for k in range(18):
                cks = jnp.full((16,), k, jnp.int32)
                ckd = jnp.full((16,), 64 + k, jnp.int32)
                acc = acc + (plsc.load_gather(b_s.at[bk], [ri, cks]) *
                             plsc.load_gather(b_d.at[bk], [ri, ckd]))
            ewb[pl.ds(g * 16, 16)] = jnp.exp(acc) + 1e-9
        pltpu.sync_copy(ewb, ew_hbm.at[pl.ds(base, CH)])
        pltpu.sync_copy(ewb, degs_sp.at[idx_s.at[bk]], add=True)
        pltpu.sync_copy(ewb, degd_sp.at[idx_d.at[bk]], add=True)
        pltpu.sync_copy(ones, ind_sp.at[idx_d.at[bk]], add=True)

    fire(0, 0)

    def pair(p, carry):
        a = 2 * p
        drain(0)
        fire(a + 1, 1)
        compute(a, 0)
        drain(1)
        fire(a + 2, 0)
        compute(a + 1, 1)
        return carry

    lax.fori_loop(0, NPAIR, pair, 0)
    drain(0)
    compute(NCHUNK - 1, 0)
    plsc.subcore_barrier()
    pltpu.sync_copy(degs_sp.at[pl.ds(s * STR, STR)],
                    degs_hbm.at[c, pl.ds(s * STR, STR)])
    pltpu.sync_copy(degd_sp.at[pl.ds(s * STR, STR)],
                    degd_hbm.at[c, pl.ds(s * STR, STR)])
    pltpu.sync_copy(ind_sp.at[pl.ds(s * STR, STR)],
                    ind_hbm.at[c, pl.ds(s * STR, STR)])


def _sc1(F, B, src, dst):
    return pl.kernel(
        _sc1_body,
        out_type=[
            jax.ShapeDtypeStruct((E,), jnp.float32),
            jax.ShapeDtypeStruct((NC, NP), jnp.float32),
            jax.ShapeDtypeStruct((NC, NP), jnp.float32),
            jax.ShapeDtypeStruct((NC, NP), jnp.float32),
        ],
        mesh=_mesh,
        compiler_params=_sc_params,
        scratch_types=[
            pltpu.VMEM((2, CH), jnp.int32),
            pltpu.VMEM((2, CH), jnp.int32),
            pltpu.VMEM((2, CH, D), jnp.float32),
            pltpu.VMEM((2, CH, D), jnp.float32),
            pltpu.VMEM((2, CH, D), jnp.float32),
            pltpu.VMEM((2, CH, D), jnp.float32),
            pltpu.VMEM((CH,), jnp.float32),
            pltpu.VMEM((CH,), jnp.float32),
            pltpu.VMEM((STR,), jnp.float32),
            pltpu.VMEM_SHARED((NP,), jnp.float32),
            pltpu.VMEM_SHARED((NP,), jnp.float32),
            pltpu.VMEM_SHARED((NP,), jnp.float32),
            pltpu.SemaphoreType.DMA,
            pltpu.SemaphoreType.DMA,
        ],
    )(F, B, src, dst)


def _tc2_body(degs_ref, degd_ref, ind_ref, beta_ref, aw_ref, theta_ref,
              ns2_ref, nd_ref, ps_ref):
    b = beta_ref[0, 0]
    betaw = 2.0 / (jnp.exp(-b) + 1.0)
    a0 = aw_ref[0, 0]
    a1 = aw_ref[0, 1]
    m = jnp.maximum(a0, a1)
    e0 = jnp.exp(a0 - m)
    e1 = jnp.exp(a1 - m)
    w0 = e0 / (e0 + e1)
    sqa = jnp.sqrt(2.0 * betaw * w0)
    degs = degs_ref[0, :] + degs_ref[1, :]
    degd = degd_ref[0, :] + degd_ref[1, :]
    ind = ind_ref[0, :] + ind_ref[1, :]
    ns2_ref[...] = lax.rsqrt(degs) / sqa
    nd_ref[...] = lax.rsqrt(degd)
    perm = 1e-9 / (jnp.exp(-theta_ref[0, 0]) + 1.0)
    ps_ref[...] = (perm / (ind + 1e-9)) / sqa


def _tc2(degs, degd, ind, beta, aw, theta):
    return pl.pallas_call(
        _tc2_body,
        out_shape=[
            jax.ShapeDtypeStruct((NP,), jnp.float32),
            jax.ShapeDtypeStruct((NP,), jnp.float32),
            jax.ShapeDtypeStruct((NP,), jnp.float32),
        ],
    )(degs, degd, ind, beta, aw, theta)


def _sc2_body(f_hbm, src_hbm, dst_hbm, ew_hbm, ns2_hbm, nd_hbm,
              rstp_hbm,
              idx_s, idx_d, rows, rows2, ewb, nsb, ndb, ab,
              rst_sp, sem0, sem1):
    c = lax.axis_index("c")
    s = lax.axis_index("s")
    wid = c * NS + s
    sems = (sem0, sem1)
    for r in range(CH):
        for cgrp in range(D // 16):
            rows2[r, pl.ds(cgrp * 16, 16)] = jnp.zeros((16,), jnp.float32)
    for j in range(8):
        off = s * 640 + j * CH

        @pl.when(off < N)
        def _():
            pltpu.sync_copy(rows2,
                            rst_sp.at[pl.ds(pl.multiple_of(off, 8), CH)])
    plsc.subcore_barrier()

    def fire(i, bk):
        base = pl.multiple_of(wid * EW + i * CH, 8)
        pltpu.sync_copy(src_hbm.at[pl.ds(base, CH)], idx_s.at[bk])
        pltpu.sync_copy(dst_hbm.at[pl.ds(base, CH)], idx_d.at[bk])
        pltpu.async_copy(f_hbm.at[idx_s.at[bk]], rows.at[bk], sems[bk])
        pltpu.async_copy(ns2_hbm.at[idx_s.at[bk]], nsb.at[bk], sems[bk])
        pltpu.async_copy(nd_hbm.at[idx_d.at[bk]], ndb.at[bk], sems[bk])
        pltpu.async_copy(ew_hbm.at[pl.ds(base, CH)], ewb.at[bk], sems[bk])

    def drain(i, bk):
        base = pl.multiple_of(wid * EW + i * CH, 8)
        pltpu.make_async_copy(f_hbm.at[idx_s.at[bk]], rows.at[bk],
                              sems[bk]).wait()
        pltpu.make_async_copy(ns2_hbm.at[idx_s.at[bk]], nsb.at[bk],
                              sems[bk]).wait()
        pltpu.make_async_copy(nd_hbm.at[idx_d.at[bk]], ndb.at[bk],
                              sems[bk]).wait()
        pltpu.make_async_copy(ew_hbm.at[pl.ds(base, CH)], ewb.at[bk],
                              sems[bk]).wait()

    def compute(i, bk):
        for g in range(G):
            sl = pl.ds(g * 16, 16)
            ab[sl] = nsb[bk, sl] * ndb[bk, sl] * ewb[bk, sl]
        for g in range(G):
            ri = lax.iota(jnp.int32, 16) + g * 16
            av = ab[pl.ds(g * 16, 16)]
            for k in range(D):
                ck = jnp.full((16,), k, jnp.int32)
                v = plsc.load_gather(rows.at[bk], [ri, ck]) * av
                plsc.store_scatter(rows2, [ri, ck], v)
        pltpu.sync_copy(rows2, rst_sp.at[idx_d.at[bk]], add=True)

    fire(0, 0)

    def pair(p, carry):
        a = 2 * p
        drain(a, 0)
        fire(a + 1, 1)
        compute(a, 0)
        drain(a + 1, 1)
        fire(a + 2, 0)
        compute(a + 1, 1)
        return carry

    lax.fori_loop(0, NPAIR, pair, 0)
    drain(NCHUNK - 1, 0)
    compute(NCHUNK - 1, 0)
    plsc.subcore_barrier()
    for j in range(8):
        off = s * 640 + j * CH

        @pl.when(off < N)
        def _():
            o8 = pl.multiple_of(off, 8)
            pltpu.sync_copy(rst_sp.at[pl.ds(o8, CH)],
                            rstp_hbm.at[pl.ds(pl.multiple_of(c * N + o8, 8),
                                              CH)])


def _sc2(F, src, dst, ew, ns2, nd):
    return pl.kernel(
        _sc2_body,
        out_type=jax.ShapeDtypeStruct((NC * N, D), jnp.float32),
        mesh=_mesh,
        compiler_params=_sc_params,
        scratch_types=[
            pltpu.VMEM((2, CH), jnp.int32),
            pltpu.VMEM((2, CH), jnp.int32),
            pltpu.VMEM((2, CH, D), jnp.float32),
            pltpu.VMEM((CH, D), jnp.float32),
            pltpu.VMEM((2, CH), jnp.float32),
            pltpu.VMEM((2, CH), jnp.float32),
            pltpu.VMEM((2, CH), jnp.float32),
            pltpu.VMEM((CH,), jnp.float32),
            pltpu.VMEM_SHARED((N, D), jnp.float32),
            pltpu.SemaphoreType.DMA,
            pltpu.SemaphoreType.DMA,
        ],
    )(F, src, dst, ew, ns2, nd)


def _tc3_body(r0_ref, r1_ref, f_ref, ps_ref, out_ref):
    x = r0_ref[...] + r1_ref[...] + f_ref[...] * ps_ref[...]
    out_ref[...] = jnp.where(x > 0, x, jnp.exp(x) - 1.0)


def _tc3(rst0, rst1, F, ps):
    BR = 1000
    return pl.pallas_call(
        _tc3_body,
        grid=(N // BR,),
        in_specs=[
            pl.BlockSpec((BR, D), lambda i: (i, 0)),
            pl.BlockSpec((BR, D), lambda i: (i, 0)),
            pl.BlockSpec((BR, D), lambda i: (i, 0)),
            pl.BlockSpec((BR, 1), lambda i: (i, 0)),
        ],
        out_specs=pl.BlockSpec((BR, D), lambda i: (i, 0)),
        out_shape=jax.ShapeDtypeStruct((N, D), jnp.float32),
    )(rst0, rst1, F, ps)


def kernel(feat, lposition, edge_index, W, w_u, w_v, w_s, beta, aw, theta):
    src = edge_index[0]
    dst = edge_index[1]
    wu = w_u.reshape(1, D)
    wv = w_v.reshape(1, D)
    ws = w_s.reshape(1, 16)
    F, B = _tc1(feat, lposition, W, wu, wv, ws, beta, aw)
    ew, degs, degd, ind = _sc1(F, B, src, dst)
    ns2, nd, ps = _tc2(degs, degd, ind, beta, aw, theta)
    rstp = _sc2(F, src, dst, ew, ns2, nd)
    out = _tc3(rstp[:N], rstp[N:], F, ps[:N].reshape(N, 1))
    return out
